# baseline (speedup 1.0000x reference)
import functools

import jax
import jax.numpy as jnp
from jax import lax
from jax.experimental import pallas as pl
from jax.experimental.pallas import tpu as pltpu
from jax.experimental.pallas import tpu_sc as plsc

NUM_F = 9
VOCAB = 100
HIDDEN = 128
NWORKERS = 32
CHUNK = 32
TAB_WORDS = NUM_F * VOCAB * HIDDEN // 2
ROW_W = HIDDEN // 2


def _tree_sum9(vals):
    s01 = vals[0] + vals[1]
    s23 = vals[2] + vals[3]
    s45 = vals[4] + vals[5]
    s67 = vals[6] + vals[7]
    a = s01 + s23
    b = s45 + s67
    return (a + b) + vals[8]


def _pack_table(tab):
    r, c = tab.shape
    t = tab.astype(jnp.bfloat16).reshape(r, c // 32, 2, 16).transpose(0, 1, 3, 2)
    return lax.bitcast_convert_type(t, jnp.int32).reshape(-1)


def _body(rows_per_worker, last_rows, nrows, x_hbm, tab_hbm, out_hbm,
          tab_v, xtv, ob0, ob1, so0, so1):
    wid = lax.axis_index("s") * 2 + lax.axis_index("c")
    base_row = wid * rows_per_worker
    is_last = wid == NWORKERS - 1

    pltpu.sync_copy(tab_hbm, tab_v)

    @pl.when(jnp.logical_not(is_last))
    def _():
        for i in range(NUM_F):
            pltpu.sync_copy(
                x_hbm.at[pl.ds(i * nrows + base_row, rows_per_worker)],
                xtv.at[pl.ds(i * rows_per_worker, rows_per_worker)])

    @pl.when(is_last)
    def _():
        for i in range(NUM_F):
            pltpu.sync_copy(
                x_hbm.at[pl.ds(i * nrows + base_row, last_rows)],
                xtv.at[pl.ds(i * rows_per_worker, last_rows)])

    my_rows = jnp.where(is_last, last_rows, rows_per_worker)
    num_chunks = my_rows // CHUNK
    npairs = num_chunks // 2

    def o_slice(c):
        return out_hbm.at[pl.ds(base_row + c * CHUNK, CHUNK), :]

    def compute(c, outbuf):
        @plsc.parallel_loop(0, CHUNK // 16, 1)
        def group_body(g):
            r0 = c * CHUNK + g * 16
            fbs = []
            for i in range(NUM_F):
                col = xtv[pl.ds(i * rows_per_worker + r0, 16)]
                fbs.append(col * ROW_W + i * (VOCAB * ROW_W))
            for r in range(16):
                bases = [fb[r] for fb in fbs]
                for sb in range(HIDDEN // 32):
                    loads = [plsc.bitcast(
                        tab_v[pl.ds(bases[i] + sb * 16, 16)], jnp.bfloat16)
                        for i in range(NUM_F)]
                    lo, hi = plsc.unpack(_tree_sum9(loads),
                                         format=plsc.PackFormat.INTERLEAVED)
                    outbuf[g * 16 + r, pl.ds(sb * 32, 16)] = lo
                    outbuf[g * 16 + r, pl.ds(sb * 32 + 16, 16)] = hi

    def pair_body(p, _):
        ca = 2 * p
        cb = ca + 1

        @pl.when(p > 0)
        def _():
            pltpu.make_async_copy(ob0, o_slice(ca), so0).wait()

        compute(ca, ob0)
        pltpu.async_copy(ob0, o_slice(ca), so0)

        @pl.when(p > 0)
        def _():
            pltpu.make_async_copy(ob1, o_slice(cb), so1).wait()

        compute(cb, ob1)
        pltpu.async_copy(ob1, o_slice(cb), so1)
        return 0

    lax.fori_loop(0, npairs, pair_body, 0)

    @pl.when(npairs > 0)
    def _():
        pltpu.make_async_copy(ob0, o_slice(0), so0).wait()
        pltpu.make_async_copy(ob1, o_slice(0), so1).wait()

    @pl.when(num_chunks % 2 == 1)
    def _():
        compute(num_chunks - 1, ob0)
        pltpu.sync_copy(ob0, o_slice(num_chunks - 1))


def kernel(x, tables):
    n = x.shape[0]
    n32 = ((n + CHUNK - 1) // CHUNK) * CHUNK
    if n32 != n:
        x = jnp.pad(x, ((0, n32 - n), (0, 0)))
    rows_per_worker = ((n32 + NWORKERS * CHUNK - 1) // (NWORKERS * CHUNK)) * CHUNK
    last_rows = n32 - (NWORKERS - 1) * rows_per_worker
    assert last_rows >= 0

    x_flat = lax.reshape(x.astype(jnp.int32), (n32 * NUM_F,), dimensions=(1, 0))
    tab_flat = _pack_table(tables.reshape(NUM_F * VOCAB, HIDDEN))

    mesh = plsc.VectorSubcoreMesh(
        core_axis_name="c", subcore_axis_name="s", num_cores=2, num_subcores=16
    )
    run = pl.kernel(
        functools.partial(_body, rows_per_worker, last_rows, n32),
        out_type=jax.ShapeDtypeStruct((n32, HIDDEN), jnp.float32),
        mesh=mesh,
        compiler_params=pltpu.CompilerParams(needs_layout_passes=False),
        scratch_types=[
            pltpu.VMEM((TAB_WORDS,), jnp.int32),
            pltpu.VMEM((NUM_F * rows_per_worker,), jnp.int32),
            pltpu.VMEM((CHUNK, HIDDEN), jnp.float32),
            pltpu.VMEM((CHUNK, HIDDEN), jnp.float32),
            pltpu.SemaphoreType.DMA,
            pltpu.SemaphoreType.DMA,
        ],
    )
    out = run(x_flat, tab_flat)
    return out[:n] if n32 != n else out


# speedup vs baseline: 2.6676x; 2.6676x over previous
import functools

import jax
import jax.numpy as jnp
from jax import lax
from jax.experimental import pallas as pl
from jax.experimental.pallas import tpu as pltpu
from jax.experimental.pallas import tpu_sc as plsc

NUM_F = 9
VOCAB = 100
HIDDEN = 128
NWORKERS = 32
CHUNK = 32
TAB_WORDS = NUM_F * VOCAB * HIDDEN // 2
ROW_W = HIDDEN // 2


def _tree_sum9(vals):
    s01 = vals[0] + vals[1]
    s23 = vals[2] + vals[3]
    s45 = vals[4] + vals[5]
    s67 = vals[6] + vals[7]
    a = s01 + s23
    b = s45 + s67
    return (a + b) + vals[8]


def _pack_table(tab):
    r, c = tab.shape
    t = tab.astype(jnp.bfloat16).reshape(r, c // 32, 2, 16).transpose(0, 1, 3, 2)
    return lax.bitcast_convert_type(t, jnp.int32).reshape(-1)


def _body(rows_per_worker, last_rows, nrows, x_hbm, tab_hbm, out_hbm,
          tab_v, xtv, xrm, ob0, ob1, so0, so1):
    wid = lax.axis_index("s") * 2 + lax.axis_index("c")
    base_row = wid * rows_per_worker
    is_last = wid == NWORKERS - 1

    pltpu.sync_copy(tab_hbm, tab_v)

    @pl.when(jnp.logical_not(is_last))
    def _():
        for i in range(NUM_F):
            pltpu.sync_copy(
                x_hbm.at[pl.ds(i * nrows + base_row, rows_per_worker)],
                xtv.at[pl.ds(i * rows_per_worker, rows_per_worker)])

    @pl.when(is_last)
    def _():
        for i in range(NUM_F):
            pltpu.sync_copy(
                x_hbm.at[pl.ds(i * nrows + base_row, last_rows)],
                xtv.at[pl.ds(i * rows_per_worker, last_rows)])

    my_rows = jnp.where(is_last, last_rows, rows_per_worker)
    num_chunks = my_rows // CHUNK
    npairs = num_chunks // 2

    iota = lax.iota(jnp.int32, 16)

    @plsc.parallel_loop(0, my_rows // 16, 1, unroll=2)
    def transpose_body(g):
        for i in range(NUM_F):
            col = xtv[pl.ds(i * rows_per_worker + g * 16, 16)]
            plsc.store_scatter(xrm, [iota * NUM_F + (g * 16 * NUM_F + i)], col)

    def o_slice(c):
        return out_hbm.at[pl.ds(base_row + c * CHUNK, CHUNK), :]

    def compute(c, outbuf):
        @plsc.parallel_loop(0, CHUNK, 1, unroll=2)
        def row_body(r):
            xv = xrm[pl.ds((c * CHUNK + r) * NUM_F, 16)]
            bases = [xv[i] * ROW_W + i * (VOCAB * ROW_W)
                     for i in range(NUM_F)]
            for sb in range(HIDDEN // 32):
                loads = [plsc.bitcast(
                    tab_v[pl.ds(bases[i] + sb * 16, 16)], jnp.bfloat16)
                    for i in range(NUM_F)]
                lo, hi = plsc.unpack(_tree_sum9(loads),
                                     format=plsc.PackFormat.INTERLEAVED)
                outbuf[r, pl.ds(sb * 32, 16)] = lo
                outbuf[r, pl.ds(sb * 32 + 16, 16)] = hi

    def pair_body(p, _):
        ca = 2 * p
        cb = ca + 1

        @pl.when(p > 0)
        def _():
            pltpu.make_async_copy(ob0, o_slice(ca), so0).wait()

        compute(ca, ob0)
        pltpu.async_copy(ob0, o_slice(ca), so0)

        @pl.when(p > 0)
        def _():
            pltpu.make_async_copy(ob1, o_slice(cb), so1).wait()

        compute(cb, ob1)
        pltpu.async_copy(ob1, o_slice(cb), so1)
        return 0

    lax.fori_loop(0, npairs, pair_body, 0)

    @pl.when(npairs > 0)
    def _():
        pltpu.make_async_copy(ob0, o_slice(0), so0).wait()
        pltpu.make_async_copy(ob1, o_slice(0), so1).wait()

    @pl.when(num_chunks % 2 == 1)
    def _():
        compute(num_chunks - 1, ob0)
        pltpu.sync_copy(ob0, o_slice(num_chunks - 1))


def kernel(x, tables):
    n = x.shape[0]
    n32 = ((n + CHUNK - 1) // CHUNK) * CHUNK
    if n32 != n:
        x = jnp.pad(x, ((0, n32 - n), (0, 0)))
    rows_per_worker = ((n32 + NWORKERS * CHUNK - 1) // (NWORKERS * CHUNK)) * CHUNK
    last_rows = n32 - (NWORKERS - 1) * rows_per_worker
    assert last_rows >= 0

    x_flat = lax.reshape(x.astype(jnp.int32), (n32 * NUM_F,), dimensions=(1, 0))
    tab_flat = _pack_table(tables.reshape(NUM_F * VOCAB, HIDDEN))

    mesh = plsc.VectorSubcoreMesh(
        core_axis_name="c", subcore_axis_name="s", num_cores=2, num_subcores=16
    )
    run = pl.kernel(
        functools.partial(_body, rows_per_worker, last_rows, n32),
        out_type=jax.ShapeDtypeStruct((n32, HIDDEN), jnp.float32),
        mesh=mesh,
        compiler_params=pltpu.CompilerParams(needs_layout_passes=False),
        scratch_types=[
            pltpu.VMEM((TAB_WORDS,), jnp.int32),
            pltpu.VMEM((NUM_F * rows_per_worker,), jnp.int32),
            pltpu.VMEM((NUM_F * rows_per_worker + 16,), jnp.int32),
            pltpu.VMEM((CHUNK, HIDDEN), jnp.float32),
            pltpu.VMEM((CHUNK, HIDDEN), jnp.float32),
            pltpu.SemaphoreType.DMA,
            pltpu.SemaphoreType.DMA,
        ],
    )
    out = run(x_flat, tab_flat)
    return out[:n] if n32 != n else out


# speedup vs baseline: 2.6706x; 1.0011x over previous
import functools

import jax
import jax.numpy as jnp
from jax import lax
from jax.experimental import pallas as pl
from jax.experimental.pallas import tpu as pltpu
from jax.experimental.pallas import tpu_sc as plsc

NUM_F = 9
VOCAB = 100
HIDDEN = 128
NWORKERS = 32
CHUNK = 32
TAB_WORDS = NUM_F * VOCAB * HIDDEN // 2
ROW_W = HIDDEN // 2


def _tree_sum9(vals):
    s01 = vals[0] + vals[1]
    s23 = vals[2] + vals[3]
    s45 = vals[4] + vals[5]
    s67 = vals[6] + vals[7]
    a = s01 + s23
    b = s45 + s67
    return (a + b) + vals[8]


def _pack_table(tab):
    r, c = tab.shape
    t = tab.astype(jnp.bfloat16).reshape(r, c // 32, 2, 16).transpose(0, 1, 3, 2)
    return lax.bitcast_convert_type(t, jnp.int32).reshape(-1)


def _body(rows_per_worker, last_rows, nrows, x_hbm, tab_hbm, out_hbm,
          tab_v, xtv, xrm, ob0, ob1, so0, so1):
    wid = lax.axis_index("s") * 2 + lax.axis_index("c")
    base_row = wid * rows_per_worker
    is_last = wid == NWORKERS - 1

    pltpu.sync_copy(tab_hbm, tab_v)

    @pl.when(jnp.logical_not(is_last))
    def _():
        for i in range(NUM_F):
            pltpu.sync_copy(
                x_hbm.at[pl.ds(i * nrows + base_row, rows_per_worker)],
                xtv.at[pl.ds(i * rows_per_worker, rows_per_worker)])

    @pl.when(is_last)
    def _():
        for i in range(NUM_F):
            pltpu.sync_copy(
                x_hbm.at[pl.ds(i * nrows + base_row, last_rows)],
                xtv.at[pl.ds(i * rows_per_worker, last_rows)])

    my_rows = jnp.where(is_last, last_rows, rows_per_worker)
    num_chunks = my_rows // CHUNK
    npairs = num_chunks // 2

    iota = lax.iota(jnp.int32, 16)

    @plsc.parallel_loop(0, my_rows // 16, 1, unroll=2)
    def transpose_body(g):
        for i in range(NUM_F):
            col = xtv[pl.ds(i * rows_per_worker + g * 16, 16)]
            plsc.store_scatter(xrm, [iota * NUM_F + (g * 16 * NUM_F + i)], col)

    def o_slice(c):
        return out_hbm.at[pl.ds(base_row + c * CHUNK, CHUNK), :]

    def compute(c, outbuf):
        @plsc.parallel_loop(0, CHUNK, 1, unroll=4)
        def row_body(r):
            xv = xrm[pl.ds((c * CHUNK + r) * NUM_F, 16)]
            bases = [xv[i] * ROW_W + i * (VOCAB * ROW_W)
                     for i in range(NUM_F)]
            for sb in range(HIDDEN // 32):
                loads = [plsc.bitcast(
                    tab_v[pl.ds(bases[i] + sb * 16, 16)], jnp.bfloat16)
                    for i in range(NUM_F)]
                lo, hi = plsc.unpack(_tree_sum9(loads),
                                     format=plsc.PackFormat.INTERLEAVED)
                outbuf[r, pl.ds(sb * 32, 16)] = lo
                outbuf[r, pl.ds(sb * 32 + 16, 16)] = hi

    def pair_body(p, _):
        ca = 2 * p
        cb = ca + 1

        @pl.when(p > 0)
        def _():
            pltpu.make_async_copy(ob0, o_slice(ca), so0).wait()

        compute(ca, ob0)
        pltpu.async_copy(ob0, o_slice(ca), so0)

        @pl.when(p > 0)
        def _():
            pltpu.make_async_copy(ob1, o_slice(cb), so1).wait()

        compute(cb, ob1)
        pltpu.async_copy(ob1, o_slice(cb), so1)
        return 0

    lax.fori_loop(0, npairs, pair_body, 0)

    @pl.when(npairs > 0)
    def _():
        pltpu.make_async_copy(ob0, o_slice(0), so0).wait()
        pltpu.make_async_copy(ob1, o_slice(0), so1).wait()

    @pl.when(num_chunks % 2 == 1)
    def _():
        compute(num_chunks - 1, ob0)
        pltpu.sync_copy(ob0, o_slice(num_chunks - 1))


def kernel(x, tables):
    n = x.shape[0]
    n32 = ((n + CHUNK - 1) // CHUNK) * CHUNK
    if n32 != n:
        x = jnp.pad(x, ((0, n32 - n), (0, 0)))
    rows_per_worker = ((n32 + NWORKERS * CHUNK - 1) // (NWORKERS * CHUNK)) * CHUNK
    last_rows = n32 - (NWORKERS - 1) * rows_per_worker
    assert last_rows >= 0

    x_flat = lax.reshape(x.astype(jnp.int32), (n32 * NUM_F,), dimensions=(1, 0))
    tab_flat = _pack_table(tables.reshape(NUM_F * VOCAB, HIDDEN))

    mesh = plsc.VectorSubcoreMesh(
        core_axis_name="c", subcore_axis_name="s", num_cores=2, num_subcores=16
    )
    run = pl.kernel(
        functools.partial(_body, rows_per_worker, last_rows, n32),
        out_type=jax.ShapeDtypeStruct((n32, HIDDEN), jnp.float32),
        mesh=mesh,
        compiler_params=pltpu.CompilerParams(needs_layout_passes=False),
        scratch_types=[
            pltpu.VMEM((TAB_WORDS,), jnp.int32),
            pltpu.VMEM((NUM_F * rows_per_worker,), jnp.int32),
            pltpu.VMEM((NUM_F * rows_per_worker + 16,), jnp.int32),
            pltpu.VMEM((CHUNK, HIDDEN), jnp.float32),
            pltpu.VMEM((CHUNK, HIDDEN), jnp.float32),
            pltpu.SemaphoreType.DMA,
            pltpu.SemaphoreType.DMA,
        ],
    )
    out = run(x_flat, tab_flat)
    return out[:n] if n32 != n else out


# speedup vs baseline: 2.8286x; 1.0592x over previous
import functools

import jax
import jax.numpy as jnp
from jax import lax
from jax.experimental import pallas as pl
from jax.experimental.pallas import tpu as pltpu
from jax.experimental.pallas import tpu_sc as plsc

NUM_F = 9
VOCAB = 100
HIDDEN = 128
NWORKERS = 32
CHUNK = 32
TAB_WORDS = NUM_F * VOCAB * HIDDEN // 2
ROW_W = HIDDEN // 2


def _tree_sum9(vals):
    s01 = vals[0] + vals[1]
    s23 = vals[2] + vals[3]
    s45 = vals[4] + vals[5]
    s67 = vals[6] + vals[7]
    a = s01 + s23
    b = s45 + s67
    return (a + b) + vals[8]


def _pack_table(tab):
    r, c = tab.shape
    t = tab.astype(jnp.bfloat16).reshape(r, c // 32, 2, 16).transpose(0, 1, 3, 2)
    return lax.bitcast_convert_type(t, jnp.int32).reshape(-1)


def _body(rows_per_worker, last_rows, nrows, x_hbm, tab_hbm, out_hbm,
          tab_v, xtv, xrm, ob0, ob1, so0, so1, st, sx):
    wid = lax.axis_index("s") * 2 + lax.axis_index("c")
    base_row = wid * rows_per_worker
    is_last = wid == NWORKERS - 1

    @pl.when(jnp.logical_not(is_last))
    def _():
        for i in range(NUM_F):
            pltpu.async_copy(
                x_hbm.at[pl.ds(i * nrows + base_row, rows_per_worker)],
                xtv.at[pl.ds(i * rows_per_worker, rows_per_worker)], sx)

    @pl.when(is_last)
    def _():
        for i in range(NUM_F):
            pltpu.async_copy(
                x_hbm.at[pl.ds(i * nrows + base_row, last_rows)],
                xtv.at[pl.ds(i * rows_per_worker, last_rows)], sx)

    pltpu.async_copy(tab_hbm, tab_v, st)

    @pl.when(jnp.logical_not(is_last))
    def _():
        for i in range(NUM_F):
            pltpu.make_async_copy(
                x_hbm.at[pl.ds(i * nrows + base_row, rows_per_worker)],
                xtv.at[pl.ds(i * rows_per_worker, rows_per_worker)], sx).wait()

    @pl.when(is_last)
    def _():
        for i in range(NUM_F):
            pltpu.make_async_copy(
                x_hbm.at[pl.ds(i * nrows + base_row, last_rows)],
                xtv.at[pl.ds(i * rows_per_worker, last_rows)], sx).wait()

    my_rows = jnp.where(is_last, last_rows, rows_per_worker)
    num_chunks = my_rows // CHUNK
    npairs = num_chunks // 2

    iota = lax.iota(jnp.int32, 16)

    @plsc.parallel_loop(0, my_rows // 16, 1, unroll=2)
    def transpose_body(g):
        for i in range(NUM_F):
            col = xtv[pl.ds(i * rows_per_worker + g * 16, 16)]
            plsc.store_scatter(xrm, [iota * NUM_F + (g * 16 * NUM_F + i)], col)

    pltpu.make_async_copy(tab_hbm, tab_v, st).wait()

    def o_slice(c):
        return out_hbm.at[pl.ds(base_row + c * CHUNK, CHUNK), :]

    def compute(c, outbuf):
        @plsc.parallel_loop(0, CHUNK, 1, unroll=4)
        def row_body(r):
            xv = xrm[pl.ds((c * CHUNK + r) * NUM_F, 16)]
            bases = [xv[i] * ROW_W + i * (VOCAB * ROW_W)
                     for i in range(NUM_F)]
            for sb in range(HIDDEN // 32):
                loads = [plsc.bitcast(
                    tab_v[pl.ds(bases[i] + sb * 16, 16)], jnp.bfloat16)
                    for i in range(NUM_F)]
                lo, hi = plsc.unpack(_tree_sum9(loads),
                                     format=plsc.PackFormat.INTERLEAVED)
                outbuf[r, pl.ds(sb * 32, 16)] = lo
                outbuf[r, pl.ds(sb * 32 + 16, 16)] = hi

    def pair_body(p, _):
        ca = 2 * p
        cb = ca + 1

        @pl.when(p > 0)
        def _():
            pltpu.make_async_copy(ob0, o_slice(ca), so0).wait()

        compute(ca, ob0)
        pltpu.async_copy(ob0, o_slice(ca), so0)

        @pl.when(p > 0)
        def _():
            pltpu.make_async_copy(ob1, o_slice(cb), so1).wait()

        compute(cb, ob1)
        pltpu.async_copy(ob1, o_slice(cb), so1)
        return 0

    lax.fori_loop(0, npairs, pair_body, 0)

    @pl.when(npairs > 0)
    def _():
        pltpu.make_async_copy(ob0, o_slice(0), so0).wait()
        pltpu.make_async_copy(ob1, o_slice(0), so1).wait()

    @pl.when(num_chunks % 2 == 1)
    def _():
        compute(num_chunks - 1, ob0)
        pltpu.sync_copy(ob0, o_slice(num_chunks - 1))


def kernel(x, tables):
    n = x.shape[0]
    n32 = ((n + CHUNK - 1) // CHUNK) * CHUNK
    if n32 != n:
        x = jnp.pad(x, ((0, n32 - n), (0, 0)))
    rows_per_worker = ((n32 + NWORKERS * CHUNK - 1) // (NWORKERS * CHUNK)) * CHUNK
    last_rows = n32 - (NWORKERS - 1) * rows_per_worker
    assert last_rows >= 0

    x_flat = lax.reshape(x.astype(jnp.int32), (n32 * NUM_F,), dimensions=(1, 0))
    tab_flat = _pack_table(tables.reshape(NUM_F * VOCAB, HIDDEN))

    mesh = plsc.VectorSubcoreMesh(
        core_axis_name="c", subcore_axis_name="s", num_cores=2, num_subcores=16
    )
    run = pl.kernel(
        functools.partial(_body, rows_per_worker, last_rows, n32),
        out_type=jax.ShapeDtypeStruct((n32, HIDDEN), jnp.float32),
        mesh=mesh,
        compiler_params=pltpu.CompilerParams(needs_layout_passes=False),
        scratch_types=[
            pltpu.VMEM((TAB_WORDS,), jnp.int32),
            pltpu.VMEM((NUM_F * rows_per_worker,), jnp.int32),
            pltpu.VMEM((NUM_F * rows_per_worker + 16,), jnp.int32),
            pltpu.VMEM((CHUNK, HIDDEN), jnp.float32),
            pltpu.VMEM((CHUNK, HIDDEN), jnp.float32),
            pltpu.SemaphoreType.DMA,
            pltpu.SemaphoreType.DMA,
            pltpu.SemaphoreType.DMA,
            pltpu.SemaphoreType.DMA,
        ],
    )
    out = run(x_flat, tab_flat)
    return out[:n] if n32 != n else out
